# baseline (speedup 1.0000x reference)
import jax
import jax.numpy as jnp
from jax.experimental import pallas as pl
from jax.experimental.pallas import tpu as pltpu

_N = 1024
_F = 64
_H = 128
_BI = 64
_Q = 256 // _BI

_PHASES = (
    (0 * _Q, 1 * _Q, 0, 1024),
    (1 * _Q, 2 * _Q, 256, 768),
    (2 * _Q, 3 * _Q, 512, 512),
    (3 * _Q, 4 * _Q, 768, 256),
)
_TRANSPOSES = (
    (256, 0, 256),
    (512, 0, 512),
    (768, 0, 768),
)


def _fused_kernel(nf, W1, b1, W2, b2, W4, b4, W5Tt, b5c, w6c, b6d,
                  out0, out1, h_s, hT_s):
    t = pl.program_id(0)

    @pl.when(t == 0)
    def _node_stage():
        h0 = nf[...]
        z = jnp.dot(h0, W1[...], preferred_element_type=jnp.float32) + b1[...]
        z = jnp.where(z >= 0.0, z, 0.01 * z)
        z = jnp.dot(z, W2[...], preferred_element_type=jnp.float32) + b2[...]
        z = jnp.where(z >= 0.0, z, 0.01 * z)
        z = jnp.dot(z, W4[...], preferred_element_type=jnp.float32) + b4[...]
        h = z + h0
        h_s[...] = h
        hT_s[...] = h.T.astype(jnp.bfloat16)

    for (lo, hi, c0, w) in _PHASES:
        @pl.when((t >= lo) & (t < hi))
        def _compute(c0=c0, w=w):
            hi_rows = h_s[pl.ds(t * _BI, _BI), :]
            hrep = jax.lax.broadcast_in_dim(hi_rows, (_BI, _H, _F), (0, 2))
            hrep = hrep.reshape(_BI * _H, _F)
            A = (hrep * W5Tt[...]).astype(jnp.bfloat16)
            T = jnp.dot(A, hT_s[:, c0:c0 + w],
                        preferred_element_type=jnp.float32)
            T = jnp.maximum(T + b5c[...], 0.0)
            U = T * w6c[...]
            D = jnp.sum(U.reshape(_BI, _H, w), axis=1) + b6d[0, 0]
            p1 = jax.nn.sigmoid(D)
            p0 = 1.0 - p1
            row = jax.lax.broadcasted_iota(jnp.int32, (_BI, w), 0)
            col = jax.lax.broadcasted_iota(jnp.int32, (_BI, w), 1)
            diag = (col + c0) == (t * _BI + row)
            r0 = t * _BI
            out0[pl.ds(r0, _BI), c0:c0 + w] = jnp.where(diag, 0.0, p0)
            out1[pl.ds(r0, _BI), c0:c0 + w] = jnp.where(diag, 0.0, p1)

    @pl.when(t == 4 * _Q)
    def _mirror():
        for (dr, dc, w) in _TRANSPOSES:
            out0[dr:dr + 256, dc:dc + w] = out0[dc:dc + w, dr:dr + 256].T
            out1[dr:dr + 256, dc:dc + w] = out1[dc:dc + w, dr:dr + 256].T


def kernel(x, node_features, W1, b1, W2, b2, W4, b4, W5, b5, W6, b6):
    f32 = jnp.float32
    W5Tt = jnp.tile(W5.T, (_BI, 1))
    b5c = jnp.tile(b5, _BI).reshape(_BI * _H, 1)
    w6c = jnp.tile(W6[:, 1] - W6[:, 0], _BI).reshape(_BI * _H, 1)
    b6d = (b6[1] - b6[0]).reshape(1, 1)

    full = lambda shape: pl.BlockSpec(shape, lambda g: tuple(0 for _ in shape))
    out0, out1 = pl.pallas_call(
        _fused_kernel,
        grid=(4 * _Q + 1,),
        in_specs=[
            full((_N, _F)),
            full((_F, _H)),
            full((1, _H)),
            full((_H, _H)),
            full((1, _H)),
            full((_H, _F)),
            full((1, _F)),
            full((_BI * _H, _F)),
            full((_BI * _H, 1)),
            full((_BI * _H, 1)),
            full((1, 1)),
        ],
        out_specs=[
            pl.BlockSpec((_N, _N), lambda g: (0, 0)),
            pl.BlockSpec((_N, _N), lambda g: (0, 0)),
        ],
        out_shape=[
            jax.ShapeDtypeStruct((_N, _N), f32),
            jax.ShapeDtypeStruct((_N, _N), f32),
        ],
        scratch_shapes=[
            pltpu.VMEM((_N, _F), f32),
            pltpu.VMEM((_F, _N), jnp.bfloat16),
        ],
    )(
        node_features, W1, b1.reshape(1, _H), W2, b2.reshape(1, _H),
        W4, b4.reshape(1, _F), W5Tt, b5c, w6c, b6d,
    )
    return jnp.stack([out0, out1], axis=-1)


# speedup vs baseline: 1.2841x; 1.2841x over previous
import jax
import jax.numpy as jnp
from jax.experimental import pallas as pl
from jax.experimental.pallas import tpu as pltpu

_N = 1024
_F = 64
_K = 72
_H = 128
_BI = 32
_Q = 128 // _BI

_PHASES = tuple(
    (p * _Q, (p + 1) * _Q, 128 * p, 1024 - 128 * p) for p in range(8)
)
_TRANSPOSES = tuple((128 * k, 128 * k) for k in range(1, 8))


def _fused_kernel(nf, W1, b1, W2, b2, W4, b4, W5Tt, w6c, b6d,
                  out0, out1, h_s, hT_s):
    t = pl.program_id(0)

    @pl.when(t == 0)
    def _node_stage():
        h0 = nf[...]
        z = jnp.dot(h0, W1[...], preferred_element_type=jnp.float32) + b1[...]
        z = jnp.where(z >= 0.0, z, 0.01 * z)
        z = jnp.dot(z, W2[...], preferred_element_type=jnp.float32) + b2[...]
        z = jnp.where(z >= 0.0, z, 0.01 * z)
        z = jnp.dot(z, W4[...], preferred_element_type=jnp.float32) + b4[...]
        h = z + h0
        h_s[:, : _F] = h
        h_s[:, _F:_K] = jnp.ones((_N, _K - _F), jnp.float32)
        hT_s[: _F, :] = h.T.astype(jnp.bfloat16)
        rid = jax.lax.broadcasted_iota(jnp.int32, (_K - _F, _N), 0)
        hT_s[_F:_K, :] = jnp.where(rid == 0, 1.0, 0.0).astype(jnp.bfloat16)

    for (lo, hi, c0, w) in _PHASES:
        @pl.when((t >= lo) & (t < hi))
        def _compute(c0=c0, w=w):
            hi_rows = h_s[pl.ds(t * _BI, _BI), :]
            hrep = jax.lax.broadcast_in_dim(hi_rows, (_BI, _H, _K), (0, 2))
            hrep = hrep.reshape(_BI * _H, _K)
            A = (hrep * W5Tt[...]).astype(jnp.bfloat16)
            T = jnp.dot(A, hT_s[:, c0:c0 + w],
                        preferred_element_type=jnp.float32)
            T = jnp.maximum(T, 0.0)
            U = T * w6c[...]
            D = jnp.sum(U.reshape(_BI, _H, w), axis=1) + b6d[0, 0]
            p1 = jax.nn.sigmoid(D)
            p0 = 1.0 - p1
            row = jax.lax.broadcasted_iota(jnp.int32, (_BI, w), 0)
            col = jax.lax.broadcasted_iota(jnp.int32, (_BI, w), 1)
            diag = (col + c0) == (t * _BI + row)
            r0 = t * _BI
            out0[pl.ds(r0, _BI), c0:c0 + w] = jnp.where(diag, 0.0, p0)
            out1[pl.ds(r0, _BI), c0:c0 + w] = jnp.where(diag, 0.0, p1)

    @pl.when(t == 8 * _Q)
    def _mirror():
        for (d, w) in _TRANSPOSES:
            out0[d:d + 128, 0:w] = out0[0:w, d:d + 128].T
            out1[d:d + 128, 0:w] = out1[0:w, d:d + 128].T


def kernel(x, node_features, W1, b1, W2, b2, W4, b4, W5, b5, W6, b6):
    f32 = jnp.float32
    W5Tt = jnp.concatenate(
        [
            jnp.tile(W5.T, (_BI, 1)),
            jnp.tile(b5, _BI).reshape(_BI * _H, 1),
            jnp.zeros((_BI * _H, _K - _F - 1), f32),
        ],
        axis=1,
    )
    w6c = jnp.tile(W6[:, 1] - W6[:, 0], _BI).reshape(_BI * _H, 1)
    b6d = (b6[1] - b6[0]).reshape(1, 1)

    full = lambda shape: pl.BlockSpec(shape, lambda g: tuple(0 for _ in shape))
    out0, out1 = pl.pallas_call(
        _fused_kernel,
        grid=(8 * _Q + 1,),
        in_specs=[
            full((_N, _F)),
            full((_F, _H)),
            full((1, _H)),
            full((_H, _H)),
            full((1, _H)),
            full((_H, _F)),
            full((1, _F)),
            full((_BI * _H, _K)),
            full((_BI * _H, 1)),
            full((1, 1)),
        ],
        out_specs=[
            pl.BlockSpec((_N, _N), lambda g: (0, 0)),
            pl.BlockSpec((_N, _N), lambda g: (0, 0)),
        ],
        out_shape=[
            jax.ShapeDtypeStruct((_N, _N), f32),
            jax.ShapeDtypeStruct((_N, _N), f32),
        ],
        scratch_shapes=[
            pltpu.VMEM((_N, _K), f32),
            pltpu.VMEM((_K, _N), jnp.bfloat16),
        ],
    )(
        node_features, W1, b1.reshape(1, _H), W2, b2.reshape(1, _H),
        W4, b4.reshape(1, _F), W5Tt, w6c, b6d,
    )
    return jnp.stack([out0, out1], axis=-1)


# speedup vs baseline: 1.3214x; 1.0291x over previous
import jax
import jax.numpy as jnp
from jax.experimental import pallas as pl
from jax.experimental.pallas import tpu as pltpu

_N = 1024
_F = 64
_K = 72
_H = 128
_BI = 32
_Q = 128 // _BI

_PHASES = tuple(
    (p * _Q, (p + 1) * _Q, 128 * p, 1024 - 128 * p) for p in range(8)
)
_TRANSPOSES = tuple((128 * k, 128 * k) for k in range(1, 8))


def _fused_kernel(nf, W1, b1, W2, b2, W4, b4, W5Tt, w6c, b6d,
                  out1, h_s, hT_s):
    t = pl.program_id(0)

    @pl.when(t == 0)
    def _node_stage():
        h0 = nf[...]
        z = jnp.dot(h0, W1[...], preferred_element_type=jnp.float32) + b1[...]
        z = jnp.where(z >= 0.0, z, 0.01 * z)
        z = jnp.dot(z, W2[...], preferred_element_type=jnp.float32) + b2[...]
        z = jnp.where(z >= 0.0, z, 0.01 * z)
        z = jnp.dot(z, W4[...], preferred_element_type=jnp.float32) + b4[...]
        h = z + h0
        h_s[:, : _F] = h
        h_s[:, _F:_K] = jnp.ones((_N, _K - _F), jnp.float32)
        hT_s[: _F, :] = h.T.astype(jnp.bfloat16)
        rid = jax.lax.broadcasted_iota(jnp.int32, (_K - _F, _N), 0)
        hT_s[_F:_K, :] = jnp.where(rid == 0, 1.0, 0.0).astype(jnp.bfloat16)

    for (lo, hi, c0, w) in _PHASES:
        @pl.when((t >= lo) & (t < hi))
        def _compute(c0=c0, w=w):
            hi_rows = h_s[pl.ds(t * _BI, _BI), :]
            hrep = jax.lax.broadcast_in_dim(hi_rows, (_BI, _H, _K), (0, 2))
            hrep = hrep.reshape(_BI * _H, _K)
            A = (hrep * W5Tt[...]).astype(jnp.bfloat16)
            T = jnp.dot(A, hT_s[:, c0:c0 + w],
                        preferred_element_type=jnp.float32)
            T = jnp.maximum(T, 0.0)
            U = T * w6c[...]
            D = jnp.sum(U.reshape(_BI, _H, w), axis=1) + b6d[0, 0]
            p1 = jax.nn.sigmoid(D)
            row = jax.lax.broadcasted_iota(jnp.int32, (_BI, w), 0)
            col = jax.lax.broadcasted_iota(jnp.int32, (_BI, w), 1)
            diag = (col + c0) == (t * _BI + row)
            r0 = t * _BI
            out1[pl.ds(r0, _BI), c0:c0 + w] = jnp.where(diag, 0.0, p1)

    @pl.when(t == 8 * _Q)
    def _mirror():
        for (d, w) in _TRANSPOSES:
            out1[d:d + 128, 0:w] = out1[0:w, d:d + 128].T


def kernel(x, node_features, W1, b1, W2, b2, W4, b4, W5, b5, W6, b6):
    f32 = jnp.float32
    W5Tt = jnp.concatenate(
        [
            jnp.tile(W5.T, (_BI, 1)),
            jnp.tile(b5, _BI).reshape(_BI * _H, 1),
            jnp.zeros((_BI * _H, _K - _F - 1), f32),
        ],
        axis=1,
    )
    w6c = jnp.tile(W6[:, 1] - W6[:, 0], _BI).reshape(_BI * _H, 1)
    b6d = (b6[1] - b6[0]).reshape(1, 1)

    full = lambda shape: pl.BlockSpec(shape, lambda g: tuple(0 for _ in shape))
    p1 = pl.pallas_call(
        _fused_kernel,
        grid=(8 * _Q + 1,),
        in_specs=[
            full((_N, _F)),
            full((_F, _H)),
            full((1, _H)),
            full((_H, _H)),
            full((1, _H)),
            full((_H, _F)),
            full((1, _F)),
            full((_BI * _H, _K)),
            full((_BI * _H, 1)),
            full((1, 1)),
        ],
        out_specs=pl.BlockSpec((_N, _N), lambda g: (0, 0)),
        out_shape=jax.ShapeDtypeStruct((_N, _N), f32),
        scratch_shapes=[
            pltpu.VMEM((_N, _K), f32),
            pltpu.VMEM((_K, _N), jnp.bfloat16),
        ],
    )(
        node_features, W1, b1.reshape(1, _H), W2, b2.reshape(1, _H),
        W4, b4.reshape(1, _F), W5Tt, w6c, b6d,
    )
    eye = (jax.lax.broadcasted_iota(jnp.int32, (_N, _N), 0)
           == jax.lax.broadcasted_iota(jnp.int32, (_N, _N), 1)).astype(f32)
    return jnp.stack([1.0 - p1 - eye, p1], axis=-1)
